# baseline (speedup 1.0000x reference)
import functools

import jax
import jax.numpy as jnp
from jax import lax
from jax.experimental import pallas as pl
from jax.experimental.pallas import tpu as pltpu
from jax.experimental.pallas import tpu_sc as plsc

NC, NS, LANES = 2, 16, 16
NW = NC * NS


def kernel(x, embedding_table, pos_table):
    B, L = x.shape
    V, D = embedding_table.shape
    BW = B // NW
    n_bg = BW // LANES

    xt = x.astype(jnp.int32).T

    mesh = plsc.VectorSubcoreMesh(
        core_axis_name="c", subcore_axis_name="s",
        num_cores=NC, num_subcores=NS,
    )

    @functools.partial(
        pl.kernel,
        out_type=jax.ShapeDtypeStruct((L, D, B), jnp.float32),
        mesh=mesh,
        scratch_types=[
            pltpu.VMEM((L, BW), jnp.int32),
            pltpu.VMEM((L, D), jnp.float32),
            pltpu.VMEM((BW, D), jnp.float32),
            pltpu.VMEM((BW, D), jnp.float32),
            pltpu.VMEM((D, BW), jnp.float32),
            pltpu.VMEM((D, BW), jnp.float32),
            pltpu.SemaphoreType.DMA,
            pltpu.SemaphoreType.DMA,
            pltpu.SemaphoreType.DMA,
            pltpu.SemaphoreType.DMA,
        ],
        compiler_params=pltpu.CompilerParams(
            use_tc_tiling_on_sc=False, needs_layout_passes=False),
    )
    def emb_kernel(idx_hbm, table_hbm, pos_hbm, out_hbm, idx_v, pos_v,
                   g0, g1, o0, o1, gsem0, gsem1, wsem0, wsem1):
        wid = lax.axis_index("s") * NC + lax.axis_index("c")
        b_base = wid * BW
        pltpu.sync_copy(idx_hbm.at[:, pl.ds(b_base, BW)], idx_v)
        pltpu.sync_copy(pos_hbm, pos_v)

        gbufs = (g0, g1)
        obufs = (o0, o1)
        gsems = (gsem0, gsem1)
        wsems = (wsem0, wsem1)

        row_ids = [lax.iota(jnp.int32, LANES) + bg * LANES for bg in range(n_bg)]

        def start_gather(l, b):
            pltpu.async_copy(table_hbm.at[idx_v.at[l]], gbufs[b], gsems[b])

        def wait_gather(b):
            pltpu.make_async_copy(
                table_hbm.at[idx_v.at[0]], gbufs[b], gsems[b]).wait()

        def start_wb(l, b):
            pltpu.async_copy(
                obufs[b], out_hbm.at[l, :, pl.ds(b_base, BW)], wsems[b])

        def wait_wb(b):
            pltpu.make_async_copy(
                obufs[b], out_hbm.at[0, :, pl.ds(0, BW)], wsems[b]).wait()

        def transpose_add(l, b):
            gbuf, obuf = gbufs[b], obufs[b]
            lvec = jnp.full((LANES,), l, jnp.int32)

            def d_body(d, carry):
                cvec = jnp.full((LANES,), d, jnp.int32)
                pvec = plsc.load_gather(pos_v, [lvec, cvec])
                for bg in range(n_bg):
                    vals = plsc.load_gather(gbuf, [row_ids[bg], cvec])
                    obuf[d, pl.ds(bg * LANES, LANES)] = vals + pvec
                return carry

            lax.fori_loop(0, D, d_body, 0)

        start_gather(0, 0)
        start_gather(1, 1)

        def l_body(io, carry):
            for k in range(2):
                l = 2 * io + k
                b = k
                wait_gather(b)

                @pl.when(l >= 2)
                def _():
                    wait_wb(b)

                transpose_add(l, b)
                start_wb(l, b)

                @pl.when(l + 2 < L)
                def _():
                    start_gather(l + 2, b)
            return carry

        lax.fori_loop(0, L // 2, l_body, 0)
        wait_wb(0)
        wait_wb(1)

    out_t = emb_kernel(xt, embedding_table, pos_table)
    return jnp.transpose(out_t, (2, 0, 1))


# speedup vs baseline: 1.1594x; 1.1594x over previous
import functools

import jax
import jax.numpy as jnp
from jax import lax
from jax.experimental import pallas as pl
from jax.experimental.pallas import tpu as pltpu
from jax.experimental.pallas import tpu_sc as plsc

NC, NS, LANES = 2, 16, 16
NW = NC * NS


def kernel(x, embedding_table, pos_table):
    B, L = x.shape
    V, D = embedding_table.shape
    BW = B // NW
    n_bg = BW // LANES

    xt = x.astype(jnp.int32).T

    mesh = plsc.VectorSubcoreMesh(
        core_axis_name="c", subcore_axis_name="s",
        num_cores=NC, num_subcores=NS,
    )

    @functools.partial(
        pl.kernel,
        out_type=jax.ShapeDtypeStruct((L, D, B), jnp.float32),
        mesh=mesh,
        scratch_types=[
            pltpu.VMEM((L, BW), jnp.int32),
            pltpu.VMEM((L, D), jnp.float32),
            pltpu.VMEM((BW, D), jnp.float32),
            pltpu.VMEM((BW, D), jnp.float32),
            pltpu.VMEM((D, BW), jnp.float32),
            pltpu.VMEM((D, BW), jnp.float32),
            pltpu.SemaphoreType.DMA,
            pltpu.SemaphoreType.DMA,
            pltpu.SemaphoreType.DMA,
            pltpu.SemaphoreType.DMA,
        ],
        compiler_params=pltpu.CompilerParams(
            use_tc_tiling_on_sc=False, needs_layout_passes=False),
    )
    def emb_kernel(idx_hbm, table_hbm, pos_hbm, out_hbm, idx_v, pos_v,
                   g0, g1, o0, o1, gsem0, gsem1, wsem0, wsem1):
        wid = lax.axis_index("s") * NC + lax.axis_index("c")
        b_base = wid * BW
        pltpu.sync_copy(idx_hbm.at[:, pl.ds(b_base, BW)], idx_v)
        pltpu.sync_copy(pos_hbm, pos_v)

        gbufs = (g0, g1)
        obufs = (o0, o1)
        gsems = (gsem0, gsem1)
        wsems = (wsem0, wsem1)

        n_dc = D // LANES
        dvecs = [lax.iota(jnp.int32, LANES) + dc * LANES for dc in range(n_dc)]

        def start_gather(l, b):
            pltpu.async_copy(table_hbm.at[idx_v.at[l]], gbufs[b], gsems[b])

        def wait_gather(b):
            pltpu.make_async_copy(
                table_hbm.at[idx_v.at[0]], gbufs[b], gsems[b]).wait()

        def start_wb(l, b):
            pltpu.async_copy(
                obufs[b], out_hbm.at[l, :, pl.ds(b_base, BW)], wsems[b])

        def wait_wb(b):
            pltpu.make_async_copy(
                obufs[b], out_hbm.at[0, :, pl.ds(0, BW)], wsems[b]).wait()

        def transpose_add(l, b):
            gbuf, obuf = gbufs[b], obufs[b]
            pos_chunks = [pos_v[l, pl.ds(dc * LANES, LANES)] for dc in range(n_dc)]

            def r_body(r, carry):
                rvec = jnp.full((LANES,), r, jnp.int32)
                for dc in range(n_dc):
                    vals = gbuf[r, pl.ds(dc * LANES, LANES)] + pos_chunks[dc]
                    plsc.store_scatter(obuf, [dvecs[dc], rvec], vals)
                return carry

            lax.fori_loop(0, BW, r_body, 0, unroll=2)

        start_gather(0, 0)
        start_gather(1, 1)

        def l_body(io, carry):
            for k in range(2):
                l = 2 * io + k
                b = k
                wait_gather(b)

                @pl.when(l >= 2)
                def _():
                    wait_wb(b)

                transpose_add(l, b)
                start_wb(l, b)

                @pl.when(l + 2 < L)
                def _():
                    start_gather(l + 2, b)
            return carry

        lax.fori_loop(0, L // 2, l_body, 0)
        wait_wb(0)
        wait_wb(1)

    out_t = emb_kernel(xt, embedding_table, pos_table)
    return jnp.transpose(out_t, (2, 0, 1))


# speedup vs baseline: 1.6522x; 1.4251x over previous
import functools

import jax
import jax.numpy as jnp
from jax import lax
from jax.experimental import pallas as pl
from jax.experimental.pallas import tpu as pltpu
from jax.experimental.pallas import tpu_sc as plsc

NC, NS, LANES = 2, 16, 16
NW = NC * NS


def kernel(x, embedding_table, pos_table):
    B, L = x.shape
    V, D = embedding_table.shape
    BW = B // NW
    n_bg = BW // LANES

    xt = x.astype(jnp.int32).T

    mesh = plsc.VectorSubcoreMesh(
        core_axis_name="c", subcore_axis_name="s",
        num_cores=NC, num_subcores=NS,
    )

    @functools.partial(
        pl.kernel,
        out_type=jax.ShapeDtypeStruct((L, D, B), jnp.float32),
        mesh=mesh,
        scratch_types=[
            pltpu.VMEM((L, BW), jnp.int32),
            pltpu.VMEM((L, D), jnp.float32),
            pltpu.VMEM((BW, D), jnp.float32),
            pltpu.VMEM((BW, D), jnp.float32),
            pltpu.VMEM((D, BW), jnp.float32),
            pltpu.VMEM((D, BW), jnp.float32),
            pltpu.SemaphoreType.DMA,
            pltpu.SemaphoreType.DMA,
            pltpu.SemaphoreType.DMA,
            pltpu.SemaphoreType.DMA,
        ],
        compiler_params=pltpu.CompilerParams(
            use_tc_tiling_on_sc=False, needs_layout_passes=False),
    )
    def emb_kernel(idx_hbm, table_hbm, pos_hbm, out_hbm, idx_v, pos_v,
                   g0, g1, o0, o1, gsem0, gsem1, wsem0, wsem1):
        wid = lax.axis_index("s") * NC + lax.axis_index("c")
        b_base = wid * BW
        pltpu.sync_copy(idx_hbm.at[:, pl.ds(b_base, BW)], idx_v)
        pltpu.sync_copy(pos_hbm, pos_v)

        gbufs = (g0, g1)
        obufs = (o0, o1)
        gsems = (gsem0, gsem1)
        wsems = (wsem0, wsem1)

        n_dc = D // LANES
        dvecs = [lax.iota(jnp.int32, LANES) + dc * LANES for dc in range(n_dc)]

        def start_gather(l, b):
            pltpu.async_copy(table_hbm.at[idx_v.at[l]], gbufs[b], gsems[b])

        def wait_gather(b):
            pltpu.make_async_copy(
                table_hbm.at[idx_v.at[0]], gbufs[b], gsems[b]).wait()

        def start_wb(l, b):
            pltpu.async_copy(
                obufs[b], out_hbm.at[l, :, pl.ds(b_base, BW)], wsems[b])

        def wait_wb(b):
            pltpu.make_async_copy(
                obufs[b], out_hbm.at[0, :, pl.ds(0, BW)], wsems[b]).wait()

        def transpose_add(l, b):
            gbuf, obuf = gbufs[b], obufs[b]
            pos_chunks = [pos_v[l, pl.ds(dc * LANES, LANES)] for dc in range(n_dc)]

            @plsc.parallel_loop(0, BW, unroll=4)
            def r_body(r):
                rvec = jnp.full((LANES,), r, jnp.int32)
                for dc in range(n_dc):
                    vals = gbuf[r, pl.ds(dc * LANES, LANES)] + pos_chunks[dc]
                    plsc.store_scatter(obuf, [dvecs[dc], rvec], vals)

        start_gather(0, 0)
        start_gather(1, 1)

        def l_body(io, carry):
            for k in range(2):
                l = 2 * io + k
                b = k
                wait_gather(b)

                @pl.when(l >= 2)
                def _():
                    wait_wb(b)

                transpose_add(l, b)
                start_wb(l, b)

                @pl.when(l + 2 < L)
                def _():
                    start_gather(l + 2, b)
            return carry

        lax.fori_loop(0, L // 2, l_body, 0)
        wait_wb(0)
        wait_wb(1)

    out_t = emb_kernel(xt, embedding_table, pos_table)
    return jnp.transpose(out_t, (2, 0, 1))


# speedup vs baseline: 6.2294x; 3.7703x over previous
import functools

import jax
import jax.numpy as jnp
from jax import lax
from jax.experimental import pallas as pl
from jax.experimental.pallas import tpu as pltpu
from jax.experimental.pallas import tpu_sc as plsc

NC, NS, LANES = 2, 16, 16
NW = NC * NS


def kernel(x, embedding_table, pos_table):
    B, L = x.shape
    V, D = embedding_table.shape
    BW = B // NW

    xt5 = (x.astype(jnp.int32).T
           .reshape(L // 8, 8, B // 128, 128)
           .transpose((0, 2, 1, 3)))

    mesh = plsc.VectorSubcoreMesh(
        core_axis_name="c", subcore_axis_name="s",
        num_cores=NC, num_subcores=NS,
    )

    n_dt = D // 8

    @functools.partial(
        pl.kernel,
        out_type=jax.ShapeDtypeStruct((L, n_dt, NW, 8, BW), jnp.float32),
        mesh=mesh,
        scratch_types=[
            pltpu.VMEM((L // 8, 8, BW), jnp.int32),
            pltpu.VMEM((L, D), jnp.float32),
            pltpu.VMEM((BW, D), jnp.float32),
            pltpu.VMEM((BW, D), jnp.float32),
            pltpu.VMEM((n_dt, 8, BW + 5), jnp.float32),
            pltpu.VMEM((n_dt, 8, BW + 5), jnp.float32),
            pltpu.SemaphoreType.DMA,
            pltpu.SemaphoreType.DMA,
            pltpu.SemaphoreType.DMA,
            pltpu.SemaphoreType.DMA,
        ],
        compiler_params=pltpu.CompilerParams(
            use_tc_tiling_on_sc=False, needs_layout_passes=False),
    )
    def emb_kernel(idx_hbm, table_hbm, pos_hbm, out_hbm, idx_v, pos_v,
                   g0, g1, o0, o1, gsem0, gsem1, wsem0, wsem1):
        wid = lax.axis_index("s") * NC + lax.axis_index("c")
        pltpu.sync_copy(idx_hbm.at[:, wid], idx_v)
        pltpu.sync_copy(pos_hbm, pos_v)

        gbufs = (g0, g1)
        obufs = (o0, o1)
        gsems = (gsem0, gsem1)
        wsems = (wsem0, wsem1)

        n_dc = D // LANES
        lane_d = lax.iota(jnp.int32, LANES)
        dtvecs = [(lane_d + dc * LANES) // 8 for dc in range(n_dc)]
        divecs = [(lane_d + dc * LANES) % 8 for dc in range(n_dc)]

        def start_gather(l, b):
            pltpu.async_copy(
                table_hbm.at[idx_v.at[l // 8, l % 8]], gbufs[b], gsems[b])

        def wait_gather(b):
            pltpu.make_async_copy(
                table_hbm.at[idx_v.at[0, 0]], gbufs[b], gsems[b]).wait()

        def start_wb(l, b):
            pltpu.async_copy(
                obufs[b].at[:, :, pl.ds(0, BW)],
                out_hbm.at[l, :, wid], wsems[b])

        def wait_wb(b):
            pltpu.make_async_copy(
                obufs[b].at[:, :, pl.ds(0, BW)],
                out_hbm.at[0, :, 0], wsems[b]).wait()

        def transpose_add(l, b):
            gbuf, obuf = gbufs[b], obufs[b]
            pos_chunks = [pos_v[l, pl.ds(dc * LANES, LANES)] for dc in range(n_dc)]

            @plsc.parallel_loop(0, BW, unroll=2)
            def r_body(r):
                rvec = jnp.full((LANES,), r, jnp.int32)
                for dc in range(n_dc):
                    vals = gbuf[r, pl.ds(dc * LANES, LANES)] + pos_chunks[dc]
                    plsc.store_scatter(obuf, [dtvecs[dc], divecs[dc], rvec], vals)

        start_gather(0, 0)
        start_gather(1, 1)

        def l_body(io, carry):
            for k in range(2):
                l = 2 * io + k
                b = k
                wait_gather(b)

                @pl.when(l >= 2)
                def _():
                    wait_wb(b)

                transpose_add(l, b)
                start_wb(l, b)

                @pl.when(l + 2 < L)
                def _():
                    start_gather(l + 2, b)
            return carry

        lax.fori_loop(0, L // 2, l_body, 0)
        wait_wb(0)
        wait_wb(1)

    out5 = emb_kernel(xt5, embedding_table, pos_table)
    return jnp.transpose(out5, (2, 4, 0, 1, 3)).reshape(B, L, D)
